# Initial kernel scaffold; baseline (speedup 1.0000x reference)
import jax
import jax.numpy as jnp
from jax.experimental import pallas as pl


def kernel(inputs, emb_table, W, b):
    raise NotImplementedError("write your pallas kernel here")


# speedup vs baseline: 165.9715x; 165.9715x over previous
import functools

import jax
import jax.numpy as jnp
from jax import lax
from jax.experimental import pallas as pl
from jax.experimental.pallas import tpu as pltpu
from jax.experimental.pallas import tpu_sc as plsc

VOCAB = 5000
VOCAB_PAD = 5008
EMBED = 16
B = 16384
L = 200

NC = 2
NS = 16
NW = NC * NS
RPW = B // NW
G = 16
NG = RPW // G


def _tw_body(table_ref, w_ref, b_ref, out_ref):
    out_ref[...] = (
        jnp.dot(table_ref[...], w_ref[...], preferred_element_type=jnp.float32)
        + b_ref[0, 0]
    )


def _compute_tw(emb_table, W, b):
    table_pad = jnp.zeros((VOCAB_PAD, EMBED), jnp.float32).at[:VOCAB].set(emb_table)
    tw = pl.pallas_call(
        _tw_body,
        out_shape=jax.ShapeDtypeStruct((VOCAB_PAD, 1), jnp.float32),
    )(table_pad, W, b.reshape(1, 1))
    return tw.reshape(VOCAB_PAD)


def _sc_body(tw_hbm, idx_hbm, out_hbm, tw_v, idx_v, out_v, sem):
    wid = lax.axis_index("c") * NS + lax.axis_index("s")
    base = wid * RPW

    pltpu.sync_copy(tw_hbm, tw_v)
    pltpu.async_copy(idx_hbm.at[pl.ds(base * L, RPW * L)], idx_v, sem).wait()

    lane = lax.iota(jnp.int32, 16) * L

    for g in range(NG):
        pos0 = lane + g * (G * L)

        def step(l, acc):
            tok = plsc.load_gather(idx_v, [pos0 + l])
            val = plsc.load_gather(tw_v, [tok])
            return acc + val

        acc = lax.fori_loop(0, L, step, jnp.zeros((16,), jnp.float32), unroll=8)
        m = acc * (1.0 / L)
        out_v[pl.ds(g * G, G)] = 1.0 / (1.0 + jnp.exp(-m))

    pltpu.sync_copy(out_v, out_hbm.at[pl.ds(base, RPW)])


@functools.partial(
    pl.kernel,
    mesh=plsc.VectorSubcoreMesh(core_axis_name="c", subcore_axis_name="s"),
    out_type=jax.ShapeDtypeStruct((B,), jnp.float32),
    scratch_types=[
        pltpu.VMEM((VOCAB_PAD,), jnp.float32),
        pltpu.VMEM((RPW * L,), jnp.int32),
        pltpu.VMEM((RPW,), jnp.float32),
        pltpu.SemaphoreType.DMA,
    ],
    compiler_params=pltpu.CompilerParams(needs_layout_passes=False),
)
def _sc_kernel(tw_hbm, idx_hbm, out_hbm, tw_v, idx_v, out_v, sem):
    _sc_body(tw_hbm, idx_hbm, out_hbm, tw_v, idx_v, out_v, sem)


def kernel(inputs, emb_table, W, b):
    tw = _compute_tw(emb_table, W, b)
    idx_flat = inputs.astype(jnp.int32).reshape(B * L)
    out = _sc_kernel(tw, idx_flat)
    return out.reshape(B, 1)
